# Initial kernel scaffold; baseline (speedup 1.0000x reference)
import jax
import jax.numpy as jnp
from jax.experimental import pallas as pl


def kernel(tensor, values, indices):
    raise NotImplementedError("write your pallas kernel here")


# speedup vs baseline: 2.5611x; 2.5611x over previous
import functools

import jax
import jax.numpy as jnp
from jax import lax
from jax.experimental import pallas as pl
from jax.experimental.pallas import tpu as pltpu
from jax.experimental.pallas import tpu_sc as plsc

_SHAPE = (4096, 11008)
_NUMEL = _SHAPE[0] * _SHAPE[1]
_K = 1000000

_NC = 2
_NS = 16
_NW = _NC * _NS
_CH = 32768
_NCHUNK = _NUMEL // _CH
_CPW = _NCHUNK // _NW
_BS = 128
_KPAD = _K + 2 * _BS
_NOFF = _NCHUNK + 1
_NOFF_PAD = 1392

_mesh = plsc.VectorSubcoreMesh(core_axis_name="c", subcore_axis_name="s")


@functools.partial(
    pl.kernel,
    out_type=jax.ShapeDtypeStruct((_NUMEL,), jnp.float32),
    mesh=_mesh,
    compiler_params=pltpu.CompilerParams(needs_layout_passes=False),
    scratch_types=[
        pltpu.VMEM((_CH,), jnp.float32),
        pltpu.VMEM((_BS,), jnp.int32),
        pltpu.VMEM((_BS,), jnp.float32),
        pltpu.VMEM((_NOFF_PAD,), jnp.int32),
    ],
)
def _scatter_merge(tensor_hbm, idx_hbm, val_hbm, off_hbm, out_hbm,
                   buf, idxb, valb, offv):
    wid = lax.axis_index("s") * _NC + lax.axis_index("c")
    pltpu.sync_copy(off_hbm, offv)

    def chunk_body(c, carry):
        cid = wid * _CPW + c
        base = cid * _CH
        pltpu.sync_copy(tensor_hbm.at[pl.ds(base, _CH)], buf)
        sev = offv[pl.ds(cid, 16)]
        s = sev[0]
        e = sev[1]
        sb0 = (s // 8) * 8
        nb = (e - sb0 + (_BS - 1)) // _BS

        def batch_body(j, bcarry):
            p0 = sb0 + j * _BS
            pltpu.sync_copy(idx_hbm.at[pl.ds(p0, _BS)], idxb)
            pltpu.sync_copy(val_hbm.at[pl.ds(p0, _BS)], valb)
            for k in range(_BS // 16):
                pos = p0 + k * 16 + lax.iota(jnp.int32, 16)
                iv = idxb[pl.ds(k * 16, 16)]
                vv = valb[pl.ds(k * 16, 16)]
                m = (pos >= s) & (pos < e)
                liv = jnp.where(m, iv - base, 0)
                plsc.addupdate_scatter(buf, [liv], vv, mask=m)
            return bcarry

        lax.fori_loop(0, nb, batch_body, 0)
        pltpu.sync_copy(buf, out_hbm.at[pl.ds(base, _CH)])
        return carry

    lax.fori_loop(0, _CPW, chunk_body, 0)


def kernel(tensor, values, indices):
    flat = tensor.reshape(-1)
    values = values.astype(jnp.float32)
    idx_p = jnp.zeros((_KPAD,), jnp.int32).at[:_K].set(indices)
    val_p = jnp.zeros((_KPAD,), jnp.float32).at[:_K].set(values)
    bounds = jnp.arange(_NOFF, dtype=jnp.int32) * _CH
    off = jnp.searchsorted(indices, bounds, side="left").astype(jnp.int32)
    off_p = jnp.zeros((_NOFF_PAD,), jnp.int32).at[:_NOFF].set(off)
    out = _scatter_merge(flat, idx_p, val_p, off_p)
    return out.reshape(_SHAPE)


# speedup vs baseline: 3.6821x; 1.4377x over previous
import functools

import jax
import jax.numpy as jnp
from jax import lax
from jax.experimental import pallas as pl
from jax.experimental.pallas import tpu as pltpu
from jax.experimental.pallas import tpu_sc as plsc

_SHAPE = (4096, 11008)
_NUMEL = _SHAPE[0] * _SHAPE[1]
_K = 1000000

_NC = 2
_NS = 16
_NW = _NC * _NS
_CH = 32768
_NCHUNK = _NUMEL // _CH
_CPW = _NCHUNK // _NW
_LB = 1024
_KPAD = _K + 2 * _LB
_NOFF = _NCHUNK + 1
_NOFF_PAD = 1392
_NIT = (_CPW + 1) // 2

_mesh = plsc.VectorSubcoreMesh(core_axis_name="c", subcore_axis_name="s")


@functools.partial(
    pl.kernel,
    out_type=jax.ShapeDtypeStruct((_NUMEL,), jnp.float32),
    mesh=_mesh,
    compiler_params=pltpu.CompilerParams(needs_layout_passes=False),
    scratch_types=[
        pltpu.VMEM((_CH,), jnp.float32),
        pltpu.VMEM((_CH,), jnp.float32),
        pltpu.VMEM((_LB,), jnp.int32),
        pltpu.VMEM((_LB,), jnp.int32),
        pltpu.VMEM((_LB,), jnp.float32),
        pltpu.VMEM((_LB,), jnp.float32),
        pltpu.VMEM((_NOFF_PAD,), jnp.int32),
        pltpu.SemaphoreType.DMA,
        pltpu.SemaphoreType.DMA,
        pltpu.SemaphoreType.DMA,
        pltpu.SemaphoreType.DMA,
    ],
)
def _scatter_merge(tensor_hbm, idx_hbm, val_hbm, off_hbm, out_hbm,
                   buf0, buf1, idx0, idx1, val0, val1, offv,
                   lsem0, lsem1, ssem0, ssem1):
    bufs = (buf0, buf1)
    idxb = (idx0, idx1)
    valb = (val0, val1)
    lsem = (lsem0, lsem1)
    ssem = (ssem0, ssem1)

    wid = lax.axis_index("s") * _NC + lax.axis_index("c")
    pltpu.sync_copy(off_hbm, offv)

    def chunk_meta(c):
        cid = wid * _CPW + c
        base = cid * _CH
        sev = offv[pl.ds(cid, 16)]
        s = sev[0]
        e = sev[1]
        sb0 = (s // 8) * 8
        return base, s, e, sb0

    def load_descs(c, slot):
        base, _, _, sb0 = chunk_meta(c)
        return (
            pltpu.make_async_copy(
                tensor_hbm.at[pl.ds(base, _CH)], bufs[slot], lsem[slot]),
            pltpu.make_async_copy(
                idx_hbm.at[pl.ds(sb0, _LB)], idxb[slot], lsem[slot]),
            pltpu.make_async_copy(
                val_hbm.at[pl.ds(sb0, _LB)], valb[slot], lsem[slot]),
        )

    def store_desc(c, slot):
        base, _, _, _ = chunk_meta(c)
        return pltpu.make_async_copy(
            bufs[slot], out_hbm.at[pl.ds(base, _CH)], ssem[slot])

    def issue_loads(c, slot):
        for d in load_descs(c, slot):
            d.start()

    def wait_loads(c, slot):
        for d in load_descs(c, slot):
            d.wait()

    def scatter_chunk(c, slot):
        base, s, e, sb0 = chunk_meta(c)
        nseg = (e - sb0 + _LB - 1) // _LB
        ibuf = idxb[slot]
        vbuf = valb[slot]
        dbuf = bufs[slot]

        def seg_body(seg, carry):
            segstart = sb0 + seg * _LB

            @pl.when(seg >= 1)
            def _():
                pltpu.sync_copy(idx_hbm.at[pl.ds(segstart, _LB)], ibuf)
                pltpu.sync_copy(val_hbm.at[pl.ds(segstart, _LB)], vbuf)

            gs = jnp.maximum(0, (s - segstart) // 16)
            ge = (jnp.minimum(e, segstart + _LB) - segstart + 15) // 16
            ge = jnp.maximum(gs, jnp.minimum(_LB // 16, ge))

            def group_body(g, gcarry):
                go = g * 16
                pos = segstart + go + lax.iota(jnp.int32, 16)
                iv = ibuf[pl.ds(go, 16)]
                vv = vbuf[pl.ds(go, 16)]
                m = (pos >= s) & (pos < e)
                liv = jnp.where(m, iv - base, 0)
                plsc.addupdate_scatter(dbuf, [liv], vv, mask=m)
                return gcarry

            lax.fori_loop(gs, ge, group_body, 0)
            return carry

        lax.fori_loop(0, nseg, seg_body, 0)

    issue_loads(0, 0)

    def pair_iter(it, carry):
        for b in range(2):
            c = it * 2 + b

            @pl.when(jnp.logical_and(c >= 1, c < _CPW - 1))
            def _():
                store_desc(c - 1, 1 - b).wait()

            @pl.when(c < _CPW - 1)
            def _():
                issue_loads(c + 1, 1 - b)

            @pl.when(c < _CPW)
            def _():
                wait_loads(c, b)
                scatter_chunk(c, b)
                store_desc(c, b).start()

        return carry

    lax.fori_loop(0, _NIT, pair_iter, 0)
    store_desc(_CPW - 2, (_CPW - 2) % 2).wait()
    store_desc(_CPW - 1, (_CPW - 1) % 2).wait()


def kernel(tensor, values, indices):
    flat = tensor.reshape(-1)
    values = values.astype(jnp.float32)
    idx_p = jnp.zeros((_KPAD,), jnp.int32).at[:_K].set(indices)
    val_p = jnp.zeros((_KPAD,), jnp.float32).at[:_K].set(values)
    bounds = jnp.arange(_NOFF, dtype=jnp.int32) * _CH
    off = jnp.searchsorted(indices, bounds, side="left").astype(jnp.int32)
    off_p = jnp.zeros((_NOFF_PAD,), jnp.int32).at[:_NOFF].set(off)
    out = _scatter_merge(flat, idx_p, val_p, off_p)
    return out.reshape(_SHAPE)
